# Initial kernel scaffold; baseline (speedup 1.0000x reference)
import functools

import jax
import jax.numpy as jnp
from jax import lax
from jax.experimental import pallas as pl
from jax.experimental.pallas import tpu as pltpu
from jax.experimental.pallas import tpu_sc as plsc

EMBED = 64
B = 4096
L = 50
TOT = B * L
NW = 32
PER_W = TOT // NW
CHUNK = 800
NCHUNK = PER_W // CHUNK

_mesh = plsc.VectorSubcoreMesh(core_axis_name="c", subcore_axis_name="s")


@functools.partial(
    pl.kernel,
    mesh=_mesh,
    out_type=jax.ShapeDtypeStruct((2, TOT, EMBED), jnp.float32),
    scratch_types=[
        pltpu.VMEM((2, NCHUNK, CHUNK), jnp.int32),
        pltpu.VMEM((CHUNK, EMBED), jnp.float32),
        pltpu.VMEM((CHUNK, EMBED), jnp.float32),
        pltpu.SemaphoreType.DMA,
        pltpu.SemaphoreType.DMA,
    ],
)
def _emb_lookup(src_idx, tgt_idx, src_tab, tgt_tab, out, idx_v, rows0, rows1,
                sem0, sem1):
    wid = lax.axis_index("s") * 2 + lax.axis_index("c")
    base = wid * PER_W

    pltpu.sync_copy(src_idx.at[wid], idx_v.at[0])
    pltpu.sync_copy(tgt_idx.at[wid], idx_v.at[1])

    bufs = (rows0, rows1)
    sems = (sem0, sem1)
    tabs = (src_tab, tgt_tab)
    chunks = [(t, j) for t in range(2) for j in range(NCHUNK)]

    def start(k):
        t, j = chunks[k]
        return pltpu.async_copy(
            tabs[t].at[idx_v.at[t, j]], bufs[k % 2], sems[k % 2])

    cop = start(0)
    for k in range(len(chunks)):
        nxt = start(k + 1) if k + 1 < len(chunks) else None
        cop.wait()
        t, j = chunks[k]
        pltpu.sync_copy(bufs[k % 2],
                        out.at[t, pl.ds(base + j * CHUNK, CHUNK)])
        cop = nxt


def kernel(src_tokens, tgt_tokens, src_table, tgt_table):
    src_idx = src_tokens.astype(jnp.int32).reshape(NW, NCHUNK, CHUNK)
    tgt_idx = tgt_tokens.astype(jnp.int32).reshape(NW, NCHUNK, CHUNK)
    out = _emb_lookup(src_idx, tgt_idx, src_table, tgt_table)
    return out.reshape(2, B, L, EMBED)


# speedup vs baseline: 4.9556x; 4.9556x over previous
import functools

import jax
import jax.numpy as jnp
from jax import lax
from jax.experimental import pallas as pl
from jax.experimental.pallas import tpu as pltpu
from jax.experimental.pallas import tpu_sc as plsc

EMBED = 64
B = 4096
L = 50
TOT = B * L
NW = 32
PER_W = TOT // NW
CHUNK = 640
NCHUNK = PER_W // CHUNK

_mesh = plsc.VectorSubcoreMesh(core_axis_name="c", subcore_axis_name="s")


NCH_TOT = 2 * NCHUNK


@functools.partial(
    pl.kernel,
    mesh=_mesh,
    out_type=jax.ShapeDtypeStruct((2, TOT, EMBED), jnp.float32),
    compiler_params=pltpu.CompilerParams(use_tc_tiling_on_sc=False),
    scratch_types=(
        [pltpu.VMEM((CHUNK,), jnp.int32) for _ in range(NCH_TOT)]
        + [
            pltpu.VMEM((CHUNK, EMBED), jnp.float32),
            pltpu.VMEM((CHUNK, EMBED), jnp.float32),
            pltpu.SemaphoreType.DMA,
            pltpu.SemaphoreType.DMA,
            pltpu.SemaphoreType.DMA,
        ]
    ),
)
def _emb_lookup(src_idx, tgt_idx, src_tab, tgt_tab, out, *scr):
    idxs = scr[:NCH_TOT]
    rows0, rows1, sem0, sem1, isem = scr[NCH_TOT:]
    wid = lax.axis_index("s") * 2 + lax.axis_index("c")
    base = wid * PER_W

    bufs = (rows0, rows1)
    sems = (sem0, sem1)
    tabs = (src_tab, tgt_tab)
    idx_hbm = (src_idx, tgt_idx)
    chunks = [(t, j) for t in range(2) for j in range(NCHUNK)]

    stages = [
        pltpu.async_copy(idx_hbm[t].at[wid, j], idxs[k], isem)
        for k, (t, j) in enumerate(chunks)
    ]
    for s in stages:
        s.wait()

    def start(k):
        t, _ = chunks[k]
        return pltpu.async_copy(tabs[t].at[idxs[k]], bufs[k % 2], sems[k % 2])

    cop = start(0)
    for k in range(NCH_TOT):
        nxt = start(k + 1) if k + 1 < NCH_TOT else None
        cop.wait()
        t, j = chunks[k]
        pltpu.sync_copy(bufs[k % 2],
                        out.at[t, pl.ds(base + j * CHUNK, CHUNK)])
        cop = nxt


def kernel(src_tokens, tgt_tokens, src_table, tgt_table):
    src_idx = src_tokens.astype(jnp.int32).reshape(NW, NCHUNK, CHUNK)
    tgt_idx = tgt_tokens.astype(jnp.int32).reshape(NW, NCHUNK, CHUNK)
    out = _emb_lookup(src_idx, tgt_idx, src_table, tgt_table)
    return out.reshape(2, B, L, EMBED)


# speedup vs baseline: 4.9632x; 1.0015x over previous
import functools

import jax
import jax.numpy as jnp
from jax import lax
from jax.experimental import pallas as pl
from jax.experimental.pallas import tpu as pltpu
from jax.experimental.pallas import tpu_sc as plsc

EMBED = 64
B = 4096
L = 50
TOT = B * L
NW = 32
PER_W = TOT // NW
CHUNK = 640
NCHUNK = PER_W // CHUNK
NCH_TOT = 2 * NCHUNK
NBUF = 3

_mesh = plsc.VectorSubcoreMesh(core_axis_name="c", subcore_axis_name="s")


@functools.partial(
    pl.kernel,
    mesh=_mesh,
    out_type=jax.ShapeDtypeStruct((2, TOT, EMBED), jnp.float32),
    compiler_params=pltpu.CompilerParams(use_tc_tiling_on_sc=False),
    scratch_types=(
        [pltpu.VMEM((CHUNK,), jnp.int32) for _ in range(NBUF)]
        + [pltpu.VMEM((CHUNK, EMBED), jnp.float32) for _ in range(NBUF)]
        + [pltpu.SemaphoreType.DMA for _ in range(2 * NBUF)]
    ),
)
def _emb_lookup(src_idx, tgt_idx, src_tab, tgt_tab, out, *scr):
    idxb = scr[:NBUF]
    rows = scr[NBUF:2 * NBUF]
    gsem = scr[2 * NBUF:3 * NBUF]
    ssem = scr[3 * NBUF:4 * NBUF]
    wid = lax.axis_index("s") * 2 + lax.axis_index("c")
    base = wid * PER_W

    tabs = (src_tab, tgt_tab)
    idx_hbm = (src_idx, tgt_idx)
    chunks = [(t, j) for t in range(2) for j in range(NCHUNK)]

    def idx_load(k):
        t, j = chunks[k]
        pltpu.sync_copy(idx_hbm[t].at[wid, j], idxb[k % NBUF])

    def start_gather(k):
        t, _ = chunks[k]
        return pltpu.async_copy(tabs[t].at[idxb[k % NBUF]], rows[k % NBUF],
                                gsem[k % NBUF])

    def start_store(k):
        t, j = chunks[k]
        return pltpu.async_copy(rows[k % NBUF],
                                out.at[t, pl.ds(base + j * CHUNK, CHUNK)],
                                ssem[k % NBUF])

    gcops = {}
    scops = {}
    for p in range(NBUF):
        idx_load(p)
        gcops[p] = start_gather(p)
    for k in range(NCH_TOT):
        gcops[k].wait()
        scops[k] = start_store(k)
        if k + NBUF < NCH_TOT:
            idx_load(k + NBUF)
            scops[k].wait()
            gcops[k + NBUF] = start_gather(k + NBUF)
        else:
            scops[k].wait()


def kernel(src_tokens, tgt_tokens, src_table, tgt_table):
    src_idx = src_tokens.astype(jnp.int32).reshape(NW, NCHUNK, CHUNK)
    tgt_idx = tgt_tokens.astype(jnp.int32).reshape(NW, NCHUNK, CHUNK)
    out = _emb_lookup(src_idx, tgt_idx, src_table, tgt_table)
    return out.reshape(2, B, L, EMBED)
